# baseline (speedup 1.0000x reference)
import jax
import jax.numpy as jnp
from jax import lax
from jax.experimental import pallas as pl
from jax.experimental.pallas import tpu as pltpu

_N_WAYS = 64
_FEAT = 64
_BK = 20000


def _knn_kernel(xs_ref, c_ref, keys_ref, labels_ref, vals_ref, valid_ref,
                am_ref, a2_ref, iota_ref, ones_ref, min_ref, arg_ref):
    step = pl.program_id(0)
    nsteps = pl.num_programs(0)

    @pl.when(step == 0)
    def _init():
        xs = xs_ref[...]
        keys = jnp.mean(xs.reshape(_N_WAYS, -1, _FEAT), axis=1)
        keys_ref[...] = keys
        am_ref[...] = -2.0 * keys
        a2_ref[...] = jnp.sum(keys * keys, axis=1, keepdims=True)
        ii = lax.broadcasted_iota(jnp.int32, (_BK, 2), 0)
        sel = lax.broadcasted_iota(jnp.int32, (_BK, 2), 1)
        iota_ref[...] = jnp.where(sel == 0, ii // 256, ii % 256
                                  ).astype(jnp.bfloat16)
        r = lax.broadcasted_iota(jnp.int32, (_FEAT, _FEAT), 0)
        c = lax.broadcasted_iota(jnp.int32, (_FEAT, _FEAT), 1)
        ones_ref[...] = (r == c).astype(jnp.float32)
        min_ref[...] = jnp.full((_N_WAYS, 1), jnp.inf, jnp.float32)
        arg_ref[...] = jnp.zeros((_N_WAYS, 1), jnp.float32)

    b = c_ref[...]
    bT = lax.dot_general(ones_ref[...], b, (((1,), (1,)), ((), ())),
                         preferred_element_type=jnp.float32)
    b2row = jnp.sum(bT * bT, axis=0, keepdims=True)
    ab = lax.dot_general(am_ref[...], b, (((1,), (1,)), ((), ())),
                         preferred_element_type=jnp.float32)
    d2 = ab + b2row
    bmin = jnp.min(d2, axis=1, keepdims=True)
    eq = (d2 == bmin).astype(jnp.bfloat16)
    barg2 = lax.dot_general(eq, iota_ref[...], (((1,), (0,)), ((), ())),
                            preferred_element_type=jnp.float32)
    barg = barg2[:, 0:1] * 256.0 + barg2[:, 1:2]
    better = bmin < min_ref[...]
    arg_ref[...] = jnp.where(better, barg + jnp.float32(step * _BK),
                             arg_ref[...])
    min_ref[...] = jnp.where(better, bmin, min_ref[...])

    @pl.when(step == nsteps - 1)
    def _fin():
        d2min = min_ref[...] + a2_ref[...]
        vals_ref[...] = jnp.sqrt(jnp.maximum(d2min, 1e-12))
        lab = arg_ref[...].astype(jnp.int32)
        labels_ref[...] = lab
        n_eq = jnp.sum((lab == lab.reshape(1, _N_WAYS)).astype(jnp.int32))
        valid_ref[...] = jnp.full((8, 128), (n_eq == _N_WAYS).astype(jnp.int32))


def kernel(combined_xs, centroid):
    nsteps = centroid.shape[0] // _BK
    class_keys, labels, vals, valid = pl.pallas_call(
        _knn_kernel,
        grid=(nsteps,),
        in_specs=[
            pl.BlockSpec(combined_xs.shape, lambda i: (0, 0)),
            pl.BlockSpec((_BK, _FEAT), lambda i: (i, 0)),
        ],
        out_specs=[
            pl.BlockSpec((_N_WAYS, _FEAT), lambda i: (0, 0)),
            pl.BlockSpec((_N_WAYS, 1), lambda i: (0, 0)),
            pl.BlockSpec((_N_WAYS, 1), lambda i: (0, 0)),
            pl.BlockSpec((8, 128), lambda i: (0, 0)),
        ],
        out_shape=[
            jax.ShapeDtypeStruct((_N_WAYS, _FEAT), jnp.float32),
            jax.ShapeDtypeStruct((_N_WAYS, 1), jnp.int32),
            jax.ShapeDtypeStruct((_N_WAYS, 1), jnp.float32),
            jax.ShapeDtypeStruct((8, 128), jnp.int32),
        ],
        scratch_shapes=[
            pltpu.VMEM((_N_WAYS, _FEAT), jnp.float32),
            pltpu.VMEM((_N_WAYS, 1), jnp.float32),
            pltpu.VMEM((_BK, 2), jnp.bfloat16),
            pltpu.VMEM((_FEAT, _FEAT), jnp.float32),
            pltpu.VMEM((_N_WAYS, 1), jnp.float32),
            pltpu.VMEM((_N_WAYS, 1), jnp.float32),
        ],
    )(combined_xs, centroid)
    return (class_keys, labels.reshape(-1), vals, valid[0, 0] != 0)


# speedup vs baseline: 1.0256x; 1.0256x over previous
import jax
import jax.numpy as jnp
from jax import lax
from jax.experimental import pallas as pl
from jax.experimental.pallas import tpu as pltpu

_N_WAYS = 64
_FEAT = 64
_BK = 25000


def _knn_kernel(xs_ref, c_ref, keys_ref, labels_ref, vals_ref, valid_ref,
                am_ref, a2_ref, iota_ref, ones_ref, min_ref, arg_ref):
    step = pl.program_id(0)
    nsteps = pl.num_programs(0)

    @pl.when(step == 0)
    def _init():
        xs = xs_ref[...]
        keys = jnp.mean(xs.reshape(_N_WAYS, -1, _FEAT), axis=1)
        keys_ref[...] = keys
        am_ref[...] = -2.0 * keys
        a2_ref[...] = jnp.sum(keys * keys, axis=1, keepdims=True)
        ii = lax.broadcasted_iota(jnp.int32, (_BK, 2), 0)
        sel = lax.broadcasted_iota(jnp.int32, (_BK, 2), 1)
        iota_ref[...] = jnp.where(sel == 0, ii // 256, ii % 256
                                  ).astype(jnp.bfloat16)
        ones_ref[...] = jnp.ones((1, _FEAT), jnp.float32)
        min_ref[...] = jnp.full((_N_WAYS, 1), jnp.inf, jnp.float32)
        arg_ref[...] = jnp.zeros((_N_WAYS, 1), jnp.float32)

    b = c_ref[...]
    b2 = jnp.sum(b * b, axis=1, keepdims=True)
    ab = lax.dot_general(am_ref[...], b, (((1,), (1,)), ((), ())),
                         preferred_element_type=jnp.float32)
    d2 = ab + b2.reshape(1, _BK)
    bmin = jnp.min(d2, axis=1, keepdims=True)
    eq = (d2 == bmin).astype(jnp.bfloat16)
    barg2 = lax.dot_general(eq, iota_ref[...], (((1,), (0,)), ((), ())),
                            preferred_element_type=jnp.float32)
    barg = barg2[:, 0:1] * 256.0 + barg2[:, 1:2]
    better = bmin < min_ref[...]
    arg_ref[...] = jnp.where(better, barg + jnp.float32(step * _BK),
                             arg_ref[...])
    min_ref[...] = jnp.where(better, bmin, min_ref[...])

    @pl.when(step == nsteps - 1)
    def _fin():
        d2min = min_ref[...] + a2_ref[...]
        vals_ref[...] = jnp.sqrt(jnp.maximum(d2min, 1e-12))
        lab = arg_ref[...].astype(jnp.int32)
        labels_ref[...] = lab
        n_eq = jnp.sum((lab == lab.reshape(1, _N_WAYS)).astype(jnp.int32))
        valid_ref[...] = jnp.full((8, 128), (n_eq == _N_WAYS).astype(jnp.int32))


def kernel(combined_xs, centroid):
    nsteps = centroid.shape[0] // _BK
    class_keys, labels, vals, valid = pl.pallas_call(
        _knn_kernel,
        grid=(nsteps,),
        in_specs=[
            pl.BlockSpec(combined_xs.shape, lambda i: (0, 0)),
            pl.BlockSpec((_BK, _FEAT), lambda i: (i, 0)),
        ],
        out_specs=[
            pl.BlockSpec((_N_WAYS, _FEAT), lambda i: (0, 0)),
            pl.BlockSpec((_N_WAYS, 1), lambda i: (0, 0)),
            pl.BlockSpec((_N_WAYS, 1), lambda i: (0, 0)),
            pl.BlockSpec((8, 128), lambda i: (0, 0)),
        ],
        out_shape=[
            jax.ShapeDtypeStruct((_N_WAYS, _FEAT), jnp.float32),
            jax.ShapeDtypeStruct((_N_WAYS, 1), jnp.int32),
            jax.ShapeDtypeStruct((_N_WAYS, 1), jnp.float32),
            jax.ShapeDtypeStruct((8, 128), jnp.int32),
        ],
        scratch_shapes=[
            pltpu.VMEM((_N_WAYS, _FEAT), jnp.float32),
            pltpu.VMEM((_N_WAYS, 1), jnp.float32),
            pltpu.VMEM((_BK, 2), jnp.bfloat16),
            pltpu.VMEM((1, _FEAT), jnp.float32),
            pltpu.VMEM((_N_WAYS, 1), jnp.float32),
            pltpu.VMEM((_N_WAYS, 1), jnp.float32),
        ],
    )(combined_xs, centroid)
    return (class_keys, labels.reshape(-1), vals, valid[0, 0] != 0)


# speedup vs baseline: 1.2692x; 1.2376x over previous
import jax
import jax.numpy as jnp
from jax import lax
from jax.experimental import pallas as pl
from jax.experimental.pallas import tpu as pltpu

_N_WAYS = 64
_FEAT = 64
_BK = 20000


def _knn_kernel(xs_ref, c_ref, keys_ref, labels_ref, vals_ref, valid_ref,
                am_ref, a2_ref, iota_ref, ones_ref, min_ref, arg_ref):
    step = pl.program_id(0)
    nsteps = pl.num_programs(0)

    @pl.when(step == 0)
    def _init():
        xs = xs_ref[...]
        keys = jnp.mean(xs.reshape(_N_WAYS, -1, _FEAT), axis=1)
        keys_ref[...] = keys
        am_ref[...] = -2.0 * keys
        a2_ref[...] = jnp.sum(keys * keys, axis=1, keepdims=True)
        ii = lax.broadcasted_iota(jnp.int32, (_BK, 2), 0)
        sel = lax.broadcasted_iota(jnp.int32, (_BK, 2), 1)
        iota_ref[...] = jnp.where(sel == 0, ii // 256, ii % 256
                                  ).astype(jnp.bfloat16)
        ones_ref[...] = jnp.ones((1, _FEAT), jnp.float32)
        min_ref[...] = jnp.full((_N_WAYS, 1), jnp.inf, jnp.float32)
        arg_ref[...] = jnp.zeros((_N_WAYS, 1), jnp.float32)

    b = c_ref[0]
    b2 = jnp.sum(b * b, axis=1, keepdims=True)
    ab = lax.dot_general(am_ref[...], b, (((1,), (1,)), ((), ())),
                         preferred_element_type=jnp.float32)
    d2 = ab + b2.reshape(1, _BK)
    bmin = jnp.min(d2, axis=1, keepdims=True)
    eq = (d2 == bmin).astype(jnp.bfloat16)
    barg2 = lax.dot_general(eq, iota_ref[...], (((1,), (0,)), ((), ())),
                            preferred_element_type=jnp.float32)
    barg = barg2[:, 0:1] * 256.0 + barg2[:, 1:2]
    better = bmin < min_ref[...]
    arg_ref[...] = jnp.where(better, barg + jnp.float32(step * _BK),
                             arg_ref[...])
    min_ref[...] = jnp.where(better, bmin, min_ref[...])

    @pl.when(step == nsteps - 1)
    def _fin():
        d2min = min_ref[...] + a2_ref[...]
        vals_ref[...] = jnp.sqrt(jnp.maximum(d2min, 1e-12))
        lab = arg_ref[...].astype(jnp.int32)
        labels_ref[...] = lab
        n_eq = jnp.sum((lab == lab.reshape(1, _N_WAYS)).astype(jnp.int32))
        valid_ref[...] = jnp.full((8, 128), (n_eq == _N_WAYS).astype(jnp.int32))


def kernel(combined_xs, centroid):
    nsteps = centroid.shape[0] // _BK
    class_keys, labels, vals, valid = pl.pallas_call(
        _knn_kernel,
        grid=(nsteps,),
        in_specs=[
            pl.BlockSpec(combined_xs.shape, lambda i: (0, 0)),
            pl.BlockSpec((1, _BK, _FEAT), lambda i: (i, 0, 0)),
        ],
        out_specs=[
            pl.BlockSpec((_N_WAYS, _FEAT), lambda i: (0, 0)),
            pl.BlockSpec((_N_WAYS, 1), lambda i: (0, 0)),
            pl.BlockSpec((_N_WAYS, 1), lambda i: (0, 0)),
            pl.BlockSpec((8, 128), lambda i: (0, 0)),
        ],
        out_shape=[
            jax.ShapeDtypeStruct((_N_WAYS, _FEAT), jnp.float32),
            jax.ShapeDtypeStruct((_N_WAYS, 1), jnp.int32),
            jax.ShapeDtypeStruct((_N_WAYS, 1), jnp.float32),
            jax.ShapeDtypeStruct((8, 128), jnp.int32),
        ],
        scratch_shapes=[
            pltpu.VMEM((_N_WAYS, _FEAT), jnp.float32),
            pltpu.VMEM((_N_WAYS, 1), jnp.float32),
            pltpu.VMEM((_BK, 2), jnp.bfloat16),
            pltpu.VMEM((1, _FEAT), jnp.float32),
            pltpu.VMEM((_N_WAYS, 1), jnp.float32),
            pltpu.VMEM((_N_WAYS, 1), jnp.float32),
        ],
    )(combined_xs, centroid.reshape(nsteps, _BK, _FEAT))
    return (class_keys, labels.reshape(-1), vals, valid[0, 0] != 0)
